# baseline (speedup 1.0000x reference)
import functools

import jax
import jax.numpy as jnp
from jax.experimental import pallas as pl
from jax.experimental.pallas import tpu as pltpu

TM = 512


def _body(w_ref, a_ref, f_ref, wp_ref, bias_ref, o_ref, x_ref, *, batch, in_f, out_f):
    @pl.when(pl.program_id(0) == 0)
    def _compute_x():
        wp = wp_ref[...]
        for bi in range(batch):
            x_ref[:, bi * out_f:(bi + 1) * out_f] = jnp.dot(
                f_ref[:, bi * in_f:(bi + 1) * in_f], wp,
                preferred_element_type=jnp.float32)

    aw = w_ref[...] * a_ref[...]
    o_ref[...] = jnp.dot(aw, x_ref[...],
                         preferred_element_type=jnp.float32) + bias_ref[...]


@jax.jit
def kernel(weights, feats, adj, W, b):
    batch, n, in_f = feats.shape
    out_f = W.shape[1]
    feats_t = feats.transpose(1, 0, 2).reshape(n, batch * in_f)
    bias_tile = jnp.tile(b, (1, batch))

    grid = (n // TM,)
    out = pl.pallas_call(
        functools.partial(_body, batch=batch, in_f=in_f, out_f=out_f),
        grid=grid,
        in_specs=[
            pl.BlockSpec((TM, n), lambda i: (i, 0)),
            pl.BlockSpec((TM, n), lambda i: (i, 0)),
            pl.BlockSpec((n, batch * in_f), lambda i: (0, 0)),
            pl.BlockSpec((in_f, out_f), lambda i: (0, 0)),
            pl.BlockSpec((1, batch * out_f), lambda i: (0, 0)),
        ],
        out_specs=pl.BlockSpec((TM, batch * out_f), lambda i: (i, 0)),
        out_shape=jax.ShapeDtypeStruct((n, batch * out_f), jnp.float32),
        scratch_shapes=[pltpu.VMEM((n, batch * out_f), jnp.float32)],
    )(weights, adj, feats_t, W, bias_tile)

    return out.reshape(n, batch, out_f).transpose(1, 0, 2)


# speedup vs baseline: 1.0941x; 1.0941x over previous
import functools

import jax
import jax.numpy as jnp
from jax.experimental import pallas as pl
from jax.experimental.pallas import tpu as pltpu

TM = 512


def _body(w_ref, a_ref, f_ref, wp_ref, bias_ref, o_ref, x_ref, *, batch, out_f):
    @pl.when(pl.program_id(0) == 0)
    def _compute_x():
        wp = wp_ref[...]
        for bi in range(batch):
            x_ref[:, bi * out_f:(bi + 1) * out_f] = jnp.dot(
                f_ref[bi], wp, preferred_element_type=jnp.float32)

    aw = w_ref[...] * a_ref[...]
    res = jnp.dot(aw, x_ref[...], preferred_element_type=jnp.float32)
    bias = bias_ref[...]
    for bi in range(batch):
        o_ref[bi] = res[:, bi * out_f:(bi + 1) * out_f] + bias


@jax.jit
def kernel(weights, feats, adj, W, b):
    batch, n, in_f = feats.shape
    out_f = W.shape[1]

    grid = (n // TM,)
    return pl.pallas_call(
        functools.partial(_body, batch=batch, out_f=out_f),
        grid=grid,
        in_specs=[
            pl.BlockSpec((TM, n), lambda i: (i, 0)),
            pl.BlockSpec((TM, n), lambda i: (i, 0)),
            pl.BlockSpec((batch, n, in_f), lambda i: (0, 0, 0)),
            pl.BlockSpec((in_f, out_f), lambda i: (0, 0)),
            pl.BlockSpec((1, out_f), lambda i: (0, 0)),
        ],
        out_specs=pl.BlockSpec((batch, TM, out_f), lambda i: (0, i, 0)),
        out_shape=jax.ShapeDtypeStruct((batch, n, out_f), jnp.float32),
        scratch_shapes=[pltpu.VMEM((n, batch * out_f), jnp.float32)],
    )(weights, adj, feats, W, b)


# speedup vs baseline: 1.1174x; 1.0213x over previous
import functools

import jax
import jax.numpy as jnp
from jax.experimental import pallas as pl
from jax.experimental.pallas import tpu as pltpu

TM = 256


def _body(w_ref, a_ref, f_ref, wp_ref, bias_ref, o_ref, x_ref, *, batch, out_f):
    @pl.when(pl.program_id(0) == 0)
    def _compute_x():
        wp = wp_ref[...]
        for bi in range(batch):
            x_ref[:, bi * out_f:(bi + 1) * out_f] = jnp.dot(
                f_ref[bi], wp, preferred_element_type=jnp.float32)

    aw = w_ref[...] * a_ref[...]
    res = jnp.dot(aw, x_ref[...], preferred_element_type=jnp.float32)
    bias = bias_ref[...]
    for bi in range(batch):
        o_ref[bi] = res[:, bi * out_f:(bi + 1) * out_f] + bias


@jax.jit
def kernel(weights, feats, adj, W, b):
    batch, n, in_f = feats.shape
    out_f = W.shape[1]

    grid = (n // TM,)
    return pl.pallas_call(
        functools.partial(_body, batch=batch, out_f=out_f),
        grid=grid,
        in_specs=[
            pl.BlockSpec((TM, n), lambda i: (i, 0)),
            pl.BlockSpec((TM, n), lambda i: (i, 0)),
            pl.BlockSpec((batch, n, in_f), lambda i: (0, 0, 0)),
            pl.BlockSpec((in_f, out_f), lambda i: (0, 0)),
            pl.BlockSpec((1, out_f), lambda i: (0, 0)),
        ],
        out_specs=pl.BlockSpec((batch, TM, out_f), lambda i: (0, i, 0)),
        out_shape=jax.ShapeDtypeStruct((batch, n, out_f), jnp.float32),
        scratch_shapes=[pltpu.VMEM((n, batch * out_f), jnp.float32)],
    )(weights, adj, feats, W, b)
